# baseline (speedup 1.0000x reference)
import functools

import jax
import jax.numpy as jnp
from jax import lax
from jax.experimental import pallas as pl
from jax.experimental.pallas import tpu as pltpu
from jax.experimental.pallas import tpu_sc as plsc

N_NODES = 10000
N_EDGES = 320000
D_FEAT = 128

_INFO = plsc.get_sparse_core_info()
NC = _INFO.num_cores
NS = _INFO.num_subcores
NW = NC * NS

CHUNK = 128
N_CHUNKS = N_EDGES // CHUNK
BASE_CHUNKS = N_CHUNKS // NW
EXTRA = N_CHUNKS - BASE_CHUNKS * NW
MAIN = (BASE_CHUNKS // 4) * 4
NSLOT = 4

ROW_BLOCKS = N_NODES // 8
RB_BASE = ROW_BLOCKS // NS
RB_EXTRA = ROW_BLOCKS - RB_BASE * NS
ROWS_BASE = RB_BASE * 8
ZCHUNK = 120
ZSPLIT = 304


@functools.partial(
    pl.kernel,
    mesh=plsc.VectorSubcoreMesh(core_axis_name="c", subcore_axis_name="s"),
    out_type=jax.ShapeDtypeStruct((NC, N_NODES, D_FEAT), jnp.float32),
    scratch_types=[
        pltpu.VMEM((NSLOT, 2, CHUNK), jnp.int32),
        pltpu.VMEM((2, CHUNK, D_FEAT), jnp.float32),
        pltpu.VMEM_SHARED((N_NODES, D_FEAT), jnp.float32),
        pltpu.SemaphoreType.DMA,
        pltpu.SemaphoreType.DMA,
        pltpu.SemaphoreType.DMA,
        pltpu.SemaphoreType.DMA,
        pltpu.SemaphoreType.DMA,
        pltpu.SemaphoreType.DMA,
        pltpu.SemaphoreType.DMA,
    ],
)
def _sc_scatter_sum(x_hbm, edges_hbm, zeros_hbm, out_hbm,
                    iv, rows_v, acc, gsem0, gsem1, is0, is1, is2, is3, zsem):
    c = lax.axis_index("c")
    s = lax.axis_index("s")
    wid = s * NC + c
    gsems = (gsem0, gsem1)
    isems = (is0, is1, is2, is3)

    row0 = (s * RB_BASE + jnp.minimum(s, RB_EXTRA)) * 8
    has_extra_rows = s < RB_EXTRA
    pltpu.async_copy(zeros_hbm, acc.at[pl.ds(row0 + ZSPLIT, ROWS_BASE - ZSPLIT)],
                     zsem)

    @pl.when(has_extra_rows)
    def _zero_extra():
        pltpu.async_copy(zeros_hbm.at[pl.ds(0, 8)],
                         acc.at[pl.ds(row0 + ROWS_BASE, 8)], zsem)

    zero16 = jnp.zeros((16,), jnp.float32)

    def _zero_row(r, carry):
        for j in range(D_FEAT // 16):
            rows_v[0, r, pl.ds(j * 16, 16)] = zero16
        return carry

    lax.fori_loop(0, ZCHUNK, _zero_row, 0)

    n_zfull = ZSPLIT // ZCHUNK
    ztail = ZSPLIT - n_zfull * ZCHUNK
    for k in range(n_zfull):
        pltpu.sync_copy(rows_v.at[0, pl.ds(0, ZCHUNK)],
                        acc.at[pl.ds(row0 + k * ZCHUNK, ZCHUNK)])
    pltpu.sync_copy(rows_v.at[0, pl.ds(0, ztail)],
                    acc.at[pl.ds(row0 + n_zfull * ZCHUNK, ztail)])

    pltpu.make_async_copy(zeros_hbm,
                          acc.at[pl.ds(row0 + ZSPLIT, ROWS_BASE - ZSPLIT)],
                          zsem).wait()

    @pl.when(has_extra_rows)
    def _wait_zero_extra():
        pltpu.make_async_copy(zeros_hbm.at[pl.ds(0, 8)],
                              acc.at[pl.ds(row0 + ROWS_BASE, 8)], zsem).wait()

    plsc.subcore_barrier()

    start = wid * BASE_CHUNKS + jnp.minimum(wid, EXTRA)
    n_chunks = BASE_CHUNKS + jnp.where(wid < EXTRA, 1, 0)

    def _fire_iload(j, q):
        pltpu.async_copy(edges_hbm.at[:, pl.ds((start + j) * CHUNK, CHUNK)],
                         iv.at[q], isems[q])

    def _wait_iload(q):
        pltpu.make_async_copy(edges_hbm.at[:, pl.ds(0, CHUNK)], iv.at[q],
                              isems[q]).wait()

    def _fire_gather(q, b):
        pltpu.async_copy(x_hbm.at[iv.at[q, 1]], rows_v.at[b], gsems[b])

    def _wait_gather(b):
        pltpu.make_async_copy(x_hbm.at[pl.ds(0, CHUNK)], rows_v.at[b],
                              gsems[b]).wait()

    def _scatter(q, b):
        pltpu.sync_copy(rows_v.at[b], acc.at[iv.at[q, 0]], add=True)

    for q in range(NSLOT):
        _fire_iload(q, q)
    for b in range(2):
        _wait_iload(b)
        _fire_gather(b, b)

    def _group4(g, carry):
        for k in range(4):
            j = g * 4 + k
            q = k
            b = k % 2
            _wait_gather(b)
            _scatter(q, b)

            @pl.when(j + NSLOT < n_chunks)
            def _refill_idx():
                _fire_iload(j + NSLOT, q)

            @pl.when(j + 2 < n_chunks)
            def _next_gather():
                _wait_iload((k + 2) % NSLOT)
                _fire_gather((k + 2) % NSLOT, b)
        return carry

    lax.fori_loop(0, MAIN // 4, _group4, 0)

    for t in range(MAIN, BASE_CHUNKS + 1):
        q = t % NSLOT
        b = t % 2

        @pl.when(t < n_chunks)
        def _tail_chunk():
            _wait_gather(b)
            _scatter(q, b)

            @pl.when(t + 2 < n_chunks)
            def _tail_gather():
                _wait_iload((t + 2) % NSLOT)
                _fire_gather((t + 2) % NSLOT, b)

    plsc.subcore_barrier()

    pltpu.sync_copy(acc.at[pl.ds(row0, ROWS_BASE)],
                    out_hbm.at[c, pl.ds(row0, ROWS_BASE)])

    @pl.when(has_extra_rows)
    def _write_extra():
        pltpu.sync_copy(acc.at[pl.ds(row0 + ROWS_BASE, 8)],
                        out_hbm.at[c, pl.ds(row0 + ROWS_BASE, 8)])


def _combine_body(p_ref, o_ref):
    o_ref[...] = p_ref[0] + p_ref[1]


_ROW_BLOCK = 2000

_combine = pl.pallas_call(
    _combine_body,
    out_shape=jax.ShapeDtypeStruct((N_NODES, D_FEAT), jnp.float32),
    grid=(N_NODES // _ROW_BLOCK,),
    in_specs=[pl.BlockSpec((NC, _ROW_BLOCK, D_FEAT), lambda i: (0, i, 0))],
    out_specs=pl.BlockSpec((_ROW_BLOCK, D_FEAT), lambda i: (i, 0)),
)


def kernel(x, edge_index):
    zeros = jnp.zeros((ROWS_BASE - ZSPLIT, D_FEAT), jnp.float32)
    partials = _sc_scatter_sum(x, edge_index.astype(jnp.int32), zeros)
    return _combine(partials)


# speedup vs baseline: 1.0107x; 1.0107x over previous
import functools

import jax
import jax.numpy as jnp
from jax import lax
from jax.experimental import pallas as pl
from jax.experimental.pallas import tpu as pltpu
from jax.experimental.pallas import tpu_sc as plsc

N_NODES = 10000
N_EDGES = 320000
D_FEAT = 128

_INFO = plsc.get_sparse_core_info()
NC = _INFO.num_cores
NS = _INFO.num_subcores
NW = NC * NS

CHUNK = 128
N_CHUNKS = N_EDGES // CHUNK
BASE_CHUNKS = N_CHUNKS // NW
EXTRA = N_CHUNKS - BASE_CHUNKS * NW
MAIN = (BASE_CHUNKS // 4) * 4
NSLOT = 4

ROW_BLOCKS = N_NODES // 8
RB_BASE = ROW_BLOCKS // NS
RB_EXTRA = ROW_BLOCKS - RB_BASE * NS
ROWS_BASE = RB_BASE * 8
ZCHUNK = 120


@functools.partial(
    pl.kernel,
    mesh=plsc.VectorSubcoreMesh(core_axis_name="c", subcore_axis_name="s"),
    out_type=jax.ShapeDtypeStruct((NC, N_NODES, D_FEAT), jnp.float32),
    scratch_types=[
        pltpu.VMEM((NSLOT, 2, CHUNK), jnp.int32),
        pltpu.VMEM((2, CHUNK, D_FEAT), jnp.float32),
        pltpu.VMEM_SHARED((N_NODES, D_FEAT), jnp.float32),
        pltpu.SemaphoreType.DMA,
        pltpu.SemaphoreType.DMA,
        pltpu.SemaphoreType.DMA,
        pltpu.SemaphoreType.DMA,
        pltpu.SemaphoreType.DMA,
        pltpu.SemaphoreType.DMA,
    ],
)
def _sc_scatter_sum(x_hbm, edges_hbm, out_hbm,
                    iv, rows_v, acc, gsem0, gsem1, is0, is1, is2, is3):
    c = lax.axis_index("c")
    s = lax.axis_index("s")
    wid = s * NC + c
    gsems = (gsem0, gsem1)
    isems = (is0, is1, is2, is3)

    zero16 = jnp.zeros((16,), jnp.float32)

    def _zero_row(r, carry):
        for j in range(D_FEAT // 16):
            rows_v[0, r, pl.ds(j * 16, 16)] = zero16
        return carry

    lax.fori_loop(0, ZCHUNK, _zero_row, 0)

    row0 = (s * RB_BASE + jnp.minimum(s, RB_EXTRA)) * 8
    has_extra_rows = s < RB_EXTRA
    n_zfull = ROWS_BASE // ZCHUNK
    ztail = ROWS_BASE - n_zfull * ZCHUNK
    for k in range(n_zfull):
        pltpu.sync_copy(rows_v.at[0, pl.ds(0, ZCHUNK)],
                        acc.at[pl.ds(row0 + k * ZCHUNK, ZCHUNK)])
    pltpu.sync_copy(rows_v.at[0, pl.ds(0, ztail)],
                    acc.at[pl.ds(row0 + n_zfull * ZCHUNK, ztail)])

    @pl.when(has_extra_rows)
    def _zero_extra():
        pltpu.sync_copy(rows_v.at[0, pl.ds(0, 8)],
                        acc.at[pl.ds(row0 + ROWS_BASE, 8)])

    plsc.subcore_barrier()

    start = wid * BASE_CHUNKS + jnp.minimum(wid, EXTRA)
    n_chunks = BASE_CHUNKS + jnp.where(wid < EXTRA, 1, 0)

    def _fire_iload(j, q):
        pltpu.async_copy(edges_hbm.at[:, pl.ds((start + j) * CHUNK, CHUNK)],
                         iv.at[q], isems[q])

    def _wait_iload(q):
        pltpu.make_async_copy(edges_hbm.at[:, pl.ds(0, CHUNK)], iv.at[q],
                              isems[q]).wait()

    def _fire_gather(q, b):
        pltpu.async_copy(x_hbm.at[iv.at[q, 1]], rows_v.at[b], gsems[b])

    def _wait_gather(b):
        pltpu.make_async_copy(x_hbm.at[pl.ds(0, CHUNK)], rows_v.at[b],
                              gsems[b]).wait()

    def _scatter(q, b):
        pltpu.sync_copy(rows_v.at[b], acc.at[iv.at[q, 0]], add=True)

    for q in range(NSLOT):
        _fire_iload(q, q)
    for b in range(2):
        _wait_iload(b)
        _fire_gather(b, b)

    def _group4(g, carry):
        for k in range(4):
            j = g * 4 + k
            q = k
            b = k % 2
            _wait_gather(b)
            _scatter(q, b)

            @pl.when(j + NSLOT < n_chunks)
            def _refill_idx():
                _fire_iload(j + NSLOT, q)

            @pl.when(j + 2 < n_chunks)
            def _next_gather():
                _wait_iload((k + 2) % NSLOT)
                _fire_gather((k + 2) % NSLOT, b)
        return carry

    lax.fori_loop(0, MAIN // 4, _group4, 0)

    for t in range(MAIN, BASE_CHUNKS + 1):
        q = t % NSLOT
        b = t % 2

        @pl.when(t < n_chunks)
        def _tail_chunk():
            _wait_gather(b)
            _scatter(q, b)

            @pl.when(t + 2 < n_chunks)
            def _tail_gather():
                _wait_iload((t + 2) % NSLOT)
                _fire_gather((t + 2) % NSLOT, b)

    plsc.subcore_barrier()

    pltpu.sync_copy(acc.at[pl.ds(row0, ROWS_BASE)],
                    out_hbm.at[c, pl.ds(row0, ROWS_BASE)])

    @pl.when(has_extra_rows)
    def _write_extra():
        pltpu.sync_copy(acc.at[pl.ds(row0 + ROWS_BASE, 8)],
                        out_hbm.at[c, pl.ds(row0 + ROWS_BASE, 8)])


def _combine_body(p_ref, o_ref):
    o_ref[...] = p_ref[0] + p_ref[1]


_ROW_BLOCK = 2000

_combine = pl.pallas_call(
    _combine_body,
    out_shape=jax.ShapeDtypeStruct((N_NODES, D_FEAT), jnp.float32),
    grid=(N_NODES // _ROW_BLOCK,),
    in_specs=[pl.BlockSpec((NC, _ROW_BLOCK, D_FEAT), lambda i: (0, i, 0))],
    out_specs=pl.BlockSpec((_ROW_BLOCK, D_FEAT), lambda i: (i, 0)),
)


def kernel(x, edge_index):
    partials = _sc_scatter_sum(x, edge_index.astype(jnp.int32))
    return _combine(partials)


# speedup vs baseline: 1.0410x; 1.0300x over previous
import functools

import jax
import jax.numpy as jnp
from jax import lax
from jax.experimental import pallas as pl
from jax.experimental.pallas import tpu as pltpu
from jax.experimental.pallas import tpu_sc as plsc

N_NODES = 10000
N_EDGES = 320000
D_FEAT = 128

_INFO = plsc.get_sparse_core_info()
NC = _INFO.num_cores
NS = _INFO.num_subcores
NW = NC * NS

CHUNK = 128
N_CHUNKS = N_EDGES // CHUNK
BASE_CHUNKS = N_CHUNKS // NW
EXTRA = N_CHUNKS - BASE_CHUNKS * NW
NBUF = 3
NSLOT = 6

ROW_BLOCKS = N_NODES // 8
RB_BASE = ROW_BLOCKS // NS
RB_EXTRA = ROW_BLOCKS - RB_BASE * NS
ROWS_BASE = RB_BASE * 8
ZCHUNK = 120


@functools.partial(
    pl.kernel,
    mesh=plsc.VectorSubcoreMesh(core_axis_name="c", subcore_axis_name="s"),
    out_type=jax.ShapeDtypeStruct((NC, N_NODES, D_FEAT), jnp.float32),
    scratch_types=[
        pltpu.VMEM((NSLOT, 2, CHUNK), jnp.int32),
        pltpu.VMEM((NBUF, CHUNK, D_FEAT), jnp.float32),
        pltpu.VMEM_SHARED((N_NODES, D_FEAT), jnp.float32),
        pltpu.SemaphoreType.DMA,
        pltpu.SemaphoreType.DMA,
        pltpu.SemaphoreType.DMA,
        pltpu.SemaphoreType.DMA,
        pltpu.SemaphoreType.DMA,
        pltpu.SemaphoreType.DMA,
        pltpu.SemaphoreType.DMA,
        pltpu.SemaphoreType.DMA,
        pltpu.SemaphoreType.DMA,
        pltpu.SemaphoreType.DMA,
        pltpu.SemaphoreType.DMA,
        pltpu.SemaphoreType.DMA,
    ],
)
def _sc_scatter_sum(x_hbm, edges_hbm, out_hbm, iv, rows_v, acc,
                    gs0, gs1, gs2, ss0, ss1, ss2,
                    is0, is1, is2, is3, is4, is5):
    c = lax.axis_index("c")
    s = lax.axis_index("s")
    wid = s * NC + c
    gsems = (gs0, gs1, gs2)
    ssems = (ss0, ss1, ss2)
    isems = (is0, is1, is2, is3, is4, is5)

    zero16 = jnp.zeros((16,), jnp.float32)

    def _zero_row(r, carry):
        for j in range(D_FEAT // 16):
            rows_v[0, r, pl.ds(j * 16, 16)] = zero16
        return carry

    lax.fori_loop(0, ZCHUNK, _zero_row, 0)

    row0 = (s * RB_BASE + jnp.minimum(s, RB_EXTRA)) * 8
    has_extra_rows = s < RB_EXTRA
    n_zfull = ROWS_BASE // ZCHUNK
    ztail = ROWS_BASE - n_zfull * ZCHUNK
    for k in range(n_zfull):
        pltpu.sync_copy(rows_v.at[0, pl.ds(0, ZCHUNK)],
                        acc.at[pl.ds(row0 + k * ZCHUNK, ZCHUNK)])
    pltpu.sync_copy(rows_v.at[0, pl.ds(0, ztail)],
                    acc.at[pl.ds(row0 + n_zfull * ZCHUNK, ztail)])

    @pl.when(has_extra_rows)
    def _zero_extra():
        pltpu.sync_copy(rows_v.at[0, pl.ds(0, 8)],
                        acc.at[pl.ds(row0 + ROWS_BASE, 8)])

    plsc.subcore_barrier()

    start = wid * BASE_CHUNKS + jnp.minimum(wid, EXTRA)
    n_chunks = BASE_CHUNKS + jnp.where(wid < EXTRA, 1, 0)

    def _fire_iload(j, q):
        pltpu.async_copy(edges_hbm.at[:, pl.ds((start + j) * CHUNK, CHUNK)],
                         iv.at[q], isems[q])

    def _wait_iload(q):
        pltpu.make_async_copy(edges_hbm.at[:, pl.ds(0, CHUNK)], iv.at[q],
                              isems[q]).wait()

    def _fire_gather(q, b):
        pltpu.async_copy(x_hbm.at[iv.at[q, 1]], rows_v.at[b], gsems[b])

    def _wait_gather(b):
        pltpu.make_async_copy(x_hbm.at[pl.ds(0, CHUNK)], rows_v.at[b],
                              gsems[b]).wait()

    def _fire_scatter(q, b):
        pltpu.async_copy(rows_v.at[b], acc.at[iv.at[q, 0]], ssems[b], add=True)

    def _wait_scatter(b):
        pltpu.make_async_copy(rows_v.at[b], acc.at[iv.at[0, 0]],
                              ssems[b]).wait()

    for q in range(4):
        _fire_iload(q, q)
    for b in range(2):
        _wait_iload(b)
        _fire_gather(b, b)

    def _group6(g, carry):
        for k in range(NSLOT):
            j = g * NSLOT + k
            b = k % NBUF
            bp = (k - 1) % NBUF
            _wait_gather(b)
            _fire_scatter(k, b)
            if k == 0:
                @pl.when(j > 0)
                def _wait_prev0():
                    _wait_scatter(bp)
            else:
                _wait_scatter(bp)

            @pl.when(j + 4 < n_chunks)
            def _refill_idx():
                _fire_iload(j + 4, (k + 4) % NSLOT)

            @pl.when(j + 2 < n_chunks)
            def _next_gather():
                _wait_iload((k + 2) % NSLOT)
                _fire_gather((k + 2) % NSLOT, (k + 2) % NBUF)
        return carry

    lax.fori_loop(0, BASE_CHUNKS // NSLOT, _group6, 0)

    @pl.when(n_chunks > BASE_CHUNKS)
    def _tail_chunk():
        _wait_gather(BASE_CHUNKS % NBUF)
        _fire_scatter(BASE_CHUNKS % NSLOT, BASE_CHUNKS % NBUF)

    _wait_scatter((BASE_CHUNKS - 1) % NBUF)

    @pl.when(n_chunks > BASE_CHUNKS)
    def _tail_drain():
        _wait_scatter(BASE_CHUNKS % NBUF)

    plsc.subcore_barrier()

    pltpu.sync_copy(acc.at[pl.ds(row0, ROWS_BASE)],
                    out_hbm.at[c, pl.ds(row0, ROWS_BASE)])

    @pl.when(has_extra_rows)
    def _write_extra():
        pltpu.sync_copy(acc.at[pl.ds(row0 + ROWS_BASE, 8)],
                        out_hbm.at[c, pl.ds(row0 + ROWS_BASE, 8)])


def _combine_body(p_ref, o_ref):
    o_ref[...] = p_ref[0] + p_ref[1]


_ROW_BLOCK = 2000

_combine = pl.pallas_call(
    _combine_body,
    out_shape=jax.ShapeDtypeStruct((N_NODES, D_FEAT), jnp.float32),
    grid=(N_NODES // _ROW_BLOCK,),
    in_specs=[pl.BlockSpec((NC, _ROW_BLOCK, D_FEAT), lambda i: (0, i, 0))],
    out_specs=pl.BlockSpec((_ROW_BLOCK, D_FEAT), lambda i: (i, 0)),
)


def kernel(x, edge_index):
    partials = _sc_scatter_sum(x, edge_index.astype(jnp.int32))
    return _combine(partials)
